# baseline (speedup 1.0000x reference)
import functools

import jax
import jax.numpy as jnp
from jax import lax
from jax.experimental import pallas as pl
from jax.experimental.pallas import tpu as pltpu, tpu_sc as plsc

_INFO = plsc.get_sparse_core_info()
_NC = _INFO.num_cores
_NS = _INFO.num_subcores
_NW = _NC * _NS
_L = _INFO.num_lanes

_B = 16384
_D = 64
_V = 1000000
_PER_W = _B // _NW
_CH = 128
_NCHUNK = _PER_W // _CH

_C = 16384
_G = -(-_V // _C)


def _matvec_body(u_idx_ref, wu_ref, wtw_ref, wtk_ref, sw_ref, sk_ref):
    u = wu_ref[u_idx_ref[0] % 8, :]
    u_bf = u.astype(jnp.bfloat16).astype(jnp.float32)
    u_lo = u_bf[:_D][:, None]
    u_hi = u_bf[_D:][:, None]
    w_bf = wtw_ref[...].astype(jnp.bfloat16).astype(jnp.float32)
    k_bf = wtk_ref[...].astype(jnp.bfloat16).astype(jnp.float32)
    sw_ref[...] = jnp.sum(w_bf * u_lo, axis=0)
    sk_ref[...] = jnp.sum(k_bf * u_hi, axis=0)


_matvec_tc = pl.pallas_call(
    _matvec_body,
    grid_spec=pltpu.PrefetchScalarGridSpec(
        num_scalar_prefetch=1,
        grid=(_G,),
        in_specs=[
            pl.BlockSpec((8, 2 * _D), lambda i, uref: (uref[0] // 8, 0)),
            pl.BlockSpec((_D, _C), lambda i, uref: (0, i)),
            pl.BlockSpec((_D, _C), lambda i, uref: (0, i)),
        ],
        out_specs=[
            pl.BlockSpec((_C,), lambda i, uref: (i,)),
            pl.BlockSpec((_C,), lambda i, uref: (i,)),
        ],
    ),
    out_shape=[
        jax.ShapeDtypeStruct((_V,), jnp.float32),
        jax.ShapeDtypeStruct((_V,), jnp.float32),
    ],
)


@functools.partial(
    pl.kernel,
    mesh=plsc.VectorSubcoreMesh(core_axis_name="c", subcore_axis_name="s"),
    compiler_params=pltpu.CompilerParams(
        needs_layout_passes=False, use_tc_tiling_on_sc=False),
    out_type=jax.ShapeDtypeStruct((_NW, _L), jnp.float32),
    scratch_types=[
        pltpu.VMEM((4, _PER_W), jnp.int32),
        pltpu.VMEM((_PER_W,), jnp.float32),
        pltpu.VMEM((_PER_W,), jnp.float32),
        pltpu.VMEM((_PER_W,), jnp.float32),
        pltpu.VMEM((_PER_W,), jnp.float32),
        pltpu.VMEM((_L,), jnp.float32),
        pltpu.SemaphoreType.DMA,
    ],
)
def _score_sc(data_hbm, sw_hbm, sk_hbm, out_hbm,
              idx_all, gwp, gkp, gwn, gkn, sum_v, sem):
    wid = lax.axis_index("s") * _NC + lax.axis_index("c")
    base = wid * _PER_W

    pltpu.sync_copy(data_hbm.at[pl.ds(1, 4), pl.ds(base, _PER_W)], idx_all)

    cps = []
    for c in range(_NCHUNK):
        sl = pl.ds(c * _CH, _CH)
        cps += [
            pltpu.async_copy(sw_hbm.at[idx_all.at[1, sl]], gwp.at[sl], sem),
            pltpu.async_copy(sk_hbm.at[idx_all.at[0, sl]], gkp.at[sl], sem),
            pltpu.async_copy(sw_hbm.at[idx_all.at[3, sl]], gwn.at[sl], sem),
            pltpu.async_copy(sk_hbm.at[idx_all.at[2, sl]], gkn.at[sl], sem),
        ]
    for cp in cps:
        cp.wait()

    one = jnp.float32(1.0)
    two = jnp.float32(2.0)

    def sigmoid(x):
        y = one + jnp.exp(-x)
        r = one / y
        r = r * (two - y * r)
        return r * (two - y * r)

    def jbody(j, total):
        sl = pl.ds(j * _L, _L)
        pos = gwp[sl] + gkp[sl]
        neg = gwn[sl] + gkn[sl]
        return total + sigmoid(pos) - sigmoid(neg)

    total = lax.fori_loop(0, _PER_W // _L, jbody,
                          jnp.zeros((_L,), jnp.float32))

    sum_v[...] = total
    pltpu.sync_copy(sum_v, out_hbm.at[wid])


def kernel(data, W_user, W_writer, W_keywd):
    data = data.astype(jnp.int32)
    u_idx = data[0, 0:1]
    s_w, s_k = _matvec_tc(u_idx, W_user, W_writer.T, W_keywd.T)
    partials = _score_sc(data, s_w, s_k)
    return jnp.sum(partials)


# speedup vs baseline: 1.1352x; 1.1352x over previous
import functools

import jax
import jax.numpy as jnp
from jax import lax
from jax.experimental import pallas as pl
from jax.experimental.pallas import tpu as pltpu, tpu_sc as plsc

_INFO = plsc.get_sparse_core_info()
_NC = _INFO.num_cores
_NS = _INFO.num_subcores
_NW = _NC * _NS
_L = _INFO.num_lanes

_B = 16384
_D = 64
_V = 1000000
_PER_W = _B // _NW
_CH = 128
_NCHUNK = _PER_W // _CH

_C = 16384
_G = -(-_V // _C)


def _matvec_body(u_idx_ref, wu_ref, wtw_ref, wtk_ref, sw_ref, sk_ref):
    u = wu_ref[u_idx_ref[0] % 8, :]
    u_bf = u.astype(jnp.bfloat16)
    w_bf = wtw_ref[...].astype(jnp.bfloat16)
    k_bf = wtk_ref[...].astype(jnp.bfloat16)
    dims = (((1,), (0,)), ((), ()))
    sw_ref[...] = jax.lax.dot_general(
        u_bf[:_D][None, :], w_bf, dims,
        preferred_element_type=jnp.float32)[0]
    sk_ref[...] = jax.lax.dot_general(
        u_bf[_D:][None, :], k_bf, dims,
        preferred_element_type=jnp.float32)[0]


_matvec_tc = pl.pallas_call(
    _matvec_body,
    grid_spec=pltpu.PrefetchScalarGridSpec(
        num_scalar_prefetch=1,
        grid=(_G,),
        in_specs=[
            pl.BlockSpec((8, 2 * _D), lambda i, uref: (uref[0] // 8, 0)),
            pl.BlockSpec((_D, _C), lambda i, uref: (0, i)),
            pl.BlockSpec((_D, _C), lambda i, uref: (0, i)),
        ],
        out_specs=[
            pl.BlockSpec((_C,), lambda i, uref: (i,)),
            pl.BlockSpec((_C,), lambda i, uref: (i,)),
        ],
    ),
    out_shape=[
        jax.ShapeDtypeStruct((_V,), jnp.float32),
        jax.ShapeDtypeStruct((_V,), jnp.float32),
    ],
)


@functools.partial(
    pl.kernel,
    mesh=plsc.VectorSubcoreMesh(core_axis_name="c", subcore_axis_name="s"),
    compiler_params=pltpu.CompilerParams(
        needs_layout_passes=False, use_tc_tiling_on_sc=False),
    out_type=jax.ShapeDtypeStruct((_NW, _L), jnp.float32),
    scratch_types=[
        pltpu.VMEM((4, _PER_W), jnp.int32),
        pltpu.VMEM((_PER_W,), jnp.float32),
        pltpu.VMEM((_PER_W,), jnp.float32),
        pltpu.VMEM((_PER_W,), jnp.float32),
        pltpu.VMEM((_PER_W,), jnp.float32),
        pltpu.VMEM((_L,), jnp.float32),
        pltpu.SemaphoreType.DMA,
    ],
)
def _score_sc(data_hbm, sw_hbm, sk_hbm, out_hbm,
              idx_all, gwp, gkp, gwn, gkn, sum_v, sem):
    wid = lax.axis_index("s") * _NC + lax.axis_index("c")
    base = wid * _PER_W

    pltpu.sync_copy(data_hbm.at[pl.ds(1, 4), pl.ds(base, _PER_W)], idx_all)

    cps = []
    for c in range(_NCHUNK):
        sl = pl.ds(c * _CH, _CH)
        cps += [
            pltpu.async_copy(sw_hbm.at[idx_all.at[1, sl]], gwp.at[sl], sem),
            pltpu.async_copy(sk_hbm.at[idx_all.at[0, sl]], gkp.at[sl], sem),
            pltpu.async_copy(sw_hbm.at[idx_all.at[3, sl]], gwn.at[sl], sem),
            pltpu.async_copy(sk_hbm.at[idx_all.at[2, sl]], gkn.at[sl], sem),
        ]
    for cp in cps:
        cp.wait()

    one = jnp.float32(1.0)
    two = jnp.float32(2.0)

    def sigmoid(x):
        y = one + jnp.exp(-x)
        r = one / y
        r = r * (two - y * r)
        return r * (two - y * r)

    def jbody(j, total):
        sl = pl.ds(j * _L, _L)
        pos = gwp[sl] + gkp[sl]
        neg = gwn[sl] + gkn[sl]
        return total + sigmoid(pos) - sigmoid(neg)

    total = lax.fori_loop(0, _PER_W // _L, jbody,
                          jnp.zeros((_L,), jnp.float32))

    sum_v[...] = total
    pltpu.sync_copy(sum_v, out_hbm.at[wid])


def kernel(data, W_user, W_writer, W_keywd):
    data = data.astype(jnp.int32)
    u_idx = data[0, 0:1]
    s_w, s_k = _matvec_tc(u_idx, W_user, W_writer.T, W_keywd.T)
    partials = _score_sc(data, s_w, s_k)
    return jnp.sum(partials)
